# baseline (speedup 1.0000x reference)
import functools
import math

import jax
import jax.numpy as jnp
from jax.experimental import pallas as pl
from jax.experimental.pallas import tpu as pltpu

_SPARSITY = 0.8


def _reduce_body(x_ref, part_ref, *, CB, HWG):
    for j in range(CB):
        blk = x_ref[pl.ds(j * HWG, HWG), :]
        part_ref[0, j, :] = jnp.sum(blk, axis=0)
        part_ref[0, CB + j, :] = jnp.max(blk, axis=0)


def _scale_body(part_ref, w1_ref, w2_ref, scale_ref, *, B, F, C, n_red, k):
    nblk, twocb, ll = part_ref.shape
    cb = twocb // 2
    ncb = nblk // (B * F)
    p = part_ref[...].reshape(B, F, ncb, 2, cb, ll)
    psum = p[:, :, :, 0]
    pmax = p[:, :, :, 1]
    avg = jnp.sum(psum, axis=(1, 4)).reshape(B, C) * (1.0 / n_red)
    mx = jnp.max(pmax, axis=(1, 4)).reshape(B, C)
    w1 = w1_ref[...]
    w2 = w2_ref[...]

    def mlp(v):
        h = jnp.sum(v[:, None, :] * w1[None, :, :], axis=-1)
        h = jnp.maximum(h, 0.0)
        return jnp.sum(h[:, None, :] * w2[None, :, :], axis=-1)

    logit = mlp(avg) + mlp(mx)
    ca = 1.0 / (1.0 + jnp.exp(-logit))

    b, c = ca.shape
    sj = ca[:, None, :]
    si = ca[:, :, None]
    ii = jax.lax.broadcasted_iota(jnp.int32, (1, c, c), 1)
    jj = jax.lax.broadcasted_iota(jnp.int32, (1, c, c), 2)
    beats = (sj > si) | ((sj == si) & (jj < ii))
    rank = jnp.sum(beats.astype(jnp.int32), axis=-1)
    scale_ref[...] = jnp.where(rank < k, ca, 0.0)[:, None, :]


def _mul_body(scale_ref, x_ref, out_ref, *, CB, HWG, NCB, F):
    i = pl.program_id(0)
    b = i // (F * NCB)
    c0 = (i % NCB) * CB
    for j in range(CB):
        s = scale_ref[b, 0, c0 + j]
        sl = pl.ds(j * HWG, HWG)
        out_ref[sl, :] = x_ref[sl, :] * s


def kernel(x, W1, W2):
    B, F, C, H, W = x.shape
    HW = H * W
    LL = 128 if HW % 128 == 0 else (8 if HW % 8 == 0 else 1)
    HWG = HW // LL
    CB = 8 if C % 8 == 0 else 1
    NCB = C // CB
    NBLK = B * F * NCB
    x2 = x.reshape(NBLK * CB * HWG, LL)
    k = int(math.ceil(C * _SPARSITY))

    x_spec = pl.BlockSpec((CB * HWG, LL), lambda i: (i, 0))

    parts = pl.pallas_call(
        functools.partial(_reduce_body, CB=CB, HWG=HWG),
        grid=(NBLK,),
        in_specs=[x_spec],
        out_specs=pl.BlockSpec((1, 2 * CB, LL), lambda i: (i, 0, 0)),
        out_shape=jax.ShapeDtypeStruct((NBLK, 2 * CB, LL), jnp.float32),
        compiler_params=pltpu.CompilerParams(
            dimension_semantics=("arbitrary",)),
    )(x2)

    scale = pl.pallas_call(
        functools.partial(_scale_body, B=B, F=F, C=C, n_red=F * HW, k=k),
        out_shape=jax.ShapeDtypeStruct((B, 1, C), jnp.float32),
    )(parts, W1, W2)

    out = pl.pallas_call(
        functools.partial(_mul_body, CB=CB, HWG=HWG, NCB=NCB, F=F),
        grid=(NBLK,),
        in_specs=[
            pl.BlockSpec(memory_space=pltpu.SMEM),
            x_spec,
        ],
        out_specs=x_spec,
        out_shape=jax.ShapeDtypeStruct((NBLK * CB * HWG, LL), jnp.float32),
        compiler_params=pltpu.CompilerParams(
            dimension_semantics=("arbitrary",)),
    )(scale, x2)
    return out.reshape(B, F, C, H, W)


# speedup vs baseline: 2.5958x; 2.5958x over previous
import functools
import math

import jax
import jax.numpy as jnp
from jax.experimental import pallas as pl
from jax.experimental.pallas import tpu as pltpu

_SPARSITY = 0.8


def _reduce_body(x_ref, sum_ref, max_ref, *, CB):
    for j in range(CB):
        blk = x_ref[0, 0, j]
        sum_ref[0, 0, j] = jnp.sum(blk, axis=0)
        max_ref[0, 0, j] = jnp.max(blk, axis=0)


def _scale_body(sum_ref, max_ref, w1_ref, w2_ref, scale_ref, *, n_red, k):
    avg = jnp.sum(sum_ref[...], axis=(1, 3)) * (1.0 / n_red)
    mx = jnp.max(jnp.max(max_ref[...], axis=1), axis=-1)
    w1 = w1_ref[...]
    w2 = w2_ref[...]

    def mlp(v):
        h = jnp.sum(v[:, None, :] * w1[None, :, :], axis=-1)
        h = jnp.maximum(h, 0.0)
        return jnp.sum(h[:, None, :] * w2[None, :, :], axis=-1)

    logit = mlp(avg) + mlp(mx)
    ca = 1.0 / (1.0 + jnp.exp(-logit))

    b, c = ca.shape
    sj = ca[:, None, :]
    si = ca[:, :, None]
    ii = jax.lax.broadcasted_iota(jnp.int32, (1, c, c), 1)
    jj = jax.lax.broadcasted_iota(jnp.int32, (1, c, c), 2)
    beats = (sj > si) | ((sj == si) & (jj < ii))
    rank = jnp.sum(beats.astype(jnp.int32), axis=-1)
    scale_ref[...] = jnp.where(rank < k, ca, 0.0)[:, None, :]


def _mul_body(scale_ref, x_ref, out_ref, *, CB):
    b = pl.program_id(0)
    i = pl.program_id(2)
    for j in range(CB):
        s = scale_ref[b, 0, i * CB + j]
        out_ref[0, 0, j] = x_ref[0, 0, j] * s


def kernel(x, W1, W2):
    B, F, C, H, W = x.shape
    CB = 8 if C % 8 == 0 else 1
    NCB = C // CB
    k = int(math.ceil(C * _SPARSITY))

    grid = (B, F, NCB)
    x_spec = pl.BlockSpec((1, 1, CB, H, W), lambda b, f, i: (b, f, i, 0, 0))
    part_spec = pl.BlockSpec((1, 1, CB, W), lambda b, f, i: (b, f, i, 0))

    sums, maxs = pl.pallas_call(
        functools.partial(_reduce_body, CB=CB),
        grid=grid,
        in_specs=[x_spec],
        out_specs=[part_spec, part_spec],
        out_shape=[jax.ShapeDtypeStruct((B, F, C, W), jnp.float32)] * 2,
        compiler_params=pltpu.CompilerParams(
            dimension_semantics=("arbitrary", "arbitrary", "arbitrary")),
    )(x)

    scale = pl.pallas_call(
        functools.partial(_scale_body, n_red=F * H * W, k=k),
        out_shape=jax.ShapeDtypeStruct((B, 1, C), jnp.float32),
    )(sums, maxs, W1, W2)

    out = pl.pallas_call(
        functools.partial(_mul_body, CB=CB),
        grid=grid,
        in_specs=[
            pl.BlockSpec(memory_space=pltpu.SMEM),
            x_spec,
        ],
        out_specs=x_spec,
        out_shape=jax.ShapeDtypeStruct((B, F, C, H, W), jnp.float32),
        compiler_params=pltpu.CompilerParams(
            dimension_semantics=("arbitrary", "arbitrary", "arbitrary")),
    )(scale, x)
    return out
